# baseline (speedup 1.0000x reference)
import functools

import jax
import jax.numpy as jnp
from jax import lax
from jax.experimental import pallas as pl
from jax.experimental.pallas import tpu as pltpu
from jax.experimental.pallas import tpu_sc as plsc

N = 50000
E = 1_600_000
NC = 2
NS = 16
NW = NC * NS
EPW = E // NW
B = 2000
NCH = EPW // B
VECS = B // 16


def _body(rx_hbm, ry_hbm, rz_hbm, ox_hbm, oy_hbm, oz_hbm, ii_hbm, ij_hbm,
          out_hbm,
          tx, ty, tz, ii_v, ij_v, gix, giy, giz, gjx, gjy, gjz,
          ofx, ofy, ofz, out_v, sem):
    c = lax.axis_index("c")
    s = lax.axis_index("s")
    wid = s * NC + c

    @pl.when(s == 0)
    def _():
        pltpu.sync_copy(rx_hbm, tx)

    @pl.when(s == 1)
    def _():
        pltpu.sync_copy(ry_hbm, ty)

    @pl.when(s == 2)
    def _():
        pltpu.sync_copy(rz_hbm, tz)

    plsc.subcore_barrier()

    iota = lax.iota(jnp.int32, 16)
    col0 = jnp.zeros((16,), jnp.int32)
    col1 = col0 + 1
    col2 = col0 + 2
    ebase = wid * EPW

    def chunk(ch, carry):
        eb = ebase + ch * B
        sl = pl.ds(eb, B)
        pltpu.sync_copy(ii_hbm.at[sl], ii_v)
        pltpu.sync_copy(ij_hbm.at[sl], ij_v)
        pltpu.sync_copy(ox_hbm.at[sl], ofx)
        pltpu.sync_copy(oy_hbm.at[sl], ofy)
        pltpu.sync_copy(oz_hbm.at[sl], ofz)

        cj1 = pltpu.async_copy(tx.at[ij_v], gjx, sem)
        cj2 = pltpu.async_copy(ty.at[ij_v], gjy, sem)
        cj3 = pltpu.async_copy(tz.at[ij_v], gjz, sem)
        ci1 = pltpu.async_copy(tx.at[ii_v], gix, sem)
        ci2 = pltpu.async_copy(ty.at[ii_v], giy, sem)
        ci3 = pltpu.async_copy(tz.at[ii_v], giz, sem)
        cj1.wait()
        cj2.wait()
        cj3.wait()
        ci1.wait()
        ci2.wait()
        ci3.wait()

        def ew(v, carry2):
            vs = pl.ds(v * 16, 16)
            rows = iota + v * 16
            plsc.store_scatter(out_v, [rows, col0], gjx[vs] - gix[vs] + ofx[vs])
            plsc.store_scatter(out_v, [rows, col1], gjy[vs] - giy[vs] + ofy[vs])
            plsc.store_scatter(out_v, [rows, col2], gjz[vs] - giz[vs] + ofz[vs])
            return carry2

        lax.fori_loop(0, VECS, ew, 0)

        pltpu.sync_copy(out_v, out_hbm.at[pl.ds(eb, B), :])
        return carry

    lax.fori_loop(0, NCH, chunk, 0)


@functools.partial(
    pl.kernel,
    mesh=plsc.VectorSubcoreMesh(core_axis_name="c", subcore_axis_name="s"),
    out_type=jax.ShapeDtypeStruct((E, 3), jnp.float32),
    compiler_params=pltpu.CompilerParams(
        needs_layout_passes=False, use_tc_tiling_on_sc=False),
    scratch_types=[
        pltpu.VMEM_SHARED((N,), jnp.float32),
        pltpu.VMEM_SHARED((N,), jnp.float32),
        pltpu.VMEM_SHARED((N,), jnp.float32),
        pltpu.VMEM((B,), jnp.int32),
        pltpu.VMEM((B,), jnp.int32),
        pltpu.VMEM((B,), jnp.float32),
        pltpu.VMEM((B,), jnp.float32),
        pltpu.VMEM((B,), jnp.float32),
        pltpu.VMEM((B,), jnp.float32),
        pltpu.VMEM((B,), jnp.float32),
        pltpu.VMEM((B,), jnp.float32),
        pltpu.VMEM((B,), jnp.float32),
        pltpu.VMEM((B,), jnp.float32),
        pltpu.VMEM((B,), jnp.float32),
        pltpu.VMEM((B, 3), jnp.float32),
        pltpu.SemaphoreType.DMA,
    ],
)
def _pairwise_sc(*refs):
    _body(*refs)


@jax.jit
def kernel(R, offsets, idx_i, idx_j):
    rx, ry, rz = R[:, 0], R[:, 1], R[:, 2]
    ox, oy, oz = offsets[:, 0], offsets[:, 1], offsets[:, 2]
    ii = idx_i.astype(jnp.int32)
    ij = idx_j.astype(jnp.int32)
    return _pairwise_sc(rx, ry, rz, ox, oy, oz, ii, ij)


# speedup vs baseline: 4.7127x; 4.7127x over previous
import functools

import jax
import jax.numpy as jnp
from jax import lax
from jax.experimental import pallas as pl
from jax.experimental.pallas import tpu as pltpu
from jax.experimental.pallas import tpu_sc as plsc

N = 50000
E = 1_600_000
NC = 2
NS = 16
NW = NC * NS
EPW = E // NW
B = 2000
NCH = EPW // B
VECS = B // 16


def _body(rx_hbm, ry_hbm, rz_hbm, ii_hbm, ij_hbm,
          outx_hbm, outy_hbm, outz_hbm,
          tx, ty, tz,
          ii0, ij0, ii1, ij1,
          gix0, giy0, giz0, gjx0, gjy0, gjz0,
          gix1, giy1, giz1, gjx1, gjy1, gjz1,
          sin0, sin1, sg0, sg1, sout0, sout1):
    c = lax.axis_index("c")
    s = lax.axis_index("s")
    wid = s * NC + c

    @pl.when(s == 0)
    def _():
        pltpu.sync_copy(rx_hbm, tx)

    @pl.when(s == 1)
    def _():
        pltpu.sync_copy(ry_hbm, ty)

    @pl.when(s == 2)
    def _():
        pltpu.sync_copy(rz_hbm, tz)

    plsc.subcore_barrier()

    ebase = wid * EPW
    ins = [(ii0, ij0, sin0), (ii1, ij1, sin1)]
    gs = [(gix0, giy0, giz0, gjx0, gjy0, gjz0, sg0),
          (gix1, giy1, giz1, gjx1, gjy1, gjz1, sg1)]
    souts = [sout0, sout1]

    def in_descs(ch):
        ii_v, ij_v, sem = ins[ch % 2]
        sl = pl.ds(ebase + ch * B, B)
        return (pltpu.make_async_copy(ii_hbm.at[sl], ii_v, sem),
                pltpu.make_async_copy(ij_hbm.at[sl], ij_v, sem))

    def g_descs(ch):
        ii_v, ij_v, _ = ins[ch % 2]
        gix, giy, giz, gjx, gjy, gjz, sem = gs[ch % 2]
        return (pltpu.make_async_copy(tx.at[ij_v], gjx, sem),
                pltpu.make_async_copy(ty.at[ij_v], gjy, sem),
                pltpu.make_async_copy(tz.at[ij_v], gjz, sem),
                pltpu.make_async_copy(tx.at[ii_v], gix, sem),
                pltpu.make_async_copy(ty.at[ii_v], giy, sem),
                pltpu.make_async_copy(tz.at[ii_v], giz, sem))

    def out_descs(ch):
        gix, giy, giz, gjx, gjy, gjz, _ = gs[ch % 2]
        sem = souts[ch % 2]
        sl = pl.ds(ebase + ch * B, B)
        return (pltpu.make_async_copy(gjx, outx_hbm.at[sl], sem),
                pltpu.make_async_copy(gjy, outy_hbm.at[sl], sem),
                pltpu.make_async_copy(gjz, outz_hbm.at[sl], sem))

    def compute(ch):
        gix, giy, giz, gjx, gjy, gjz, _ = gs[ch % 2]

        def ew(v, carry):
            vs = pl.ds(v * 16, 16)
            gjx[vs] = gjx[vs] - gix[vs]
            gjy[vs] = gjy[vs] - giy[vs]
            gjz[vs] = gjz[vs] - giz[vs]
            return carry

        lax.fori_loop(0, VECS, ew, 0, unroll=5)

    for d in in_descs(0):
        d.start()
    for d in in_descs(0):
        d.wait()
    for d in g_descs(0):
        d.start()
    if NCH > 1:
        for d in in_descs(1):
            d.start()

    for ch in range(NCH):
        if ch + 1 < NCH:
            for d in in_descs(ch + 1):
                d.wait()
            if ch >= 1:
                for d in out_descs(ch - 1):
                    d.wait()
            for d in g_descs(ch + 1):
                d.start()
        for d in g_descs(ch):
            d.wait()
        if ch + 2 < NCH:
            for d in in_descs(ch + 2):
                d.start()
        compute(ch)
        for d in out_descs(ch):
            d.start()

    for d in out_descs(NCH - 2):
        d.wait()
    for d in out_descs(NCH - 1):
        d.wait()


@functools.partial(
    pl.kernel,
    mesh=plsc.VectorSubcoreMesh(core_axis_name="c", subcore_axis_name="s"),
    out_type=(
        jax.ShapeDtypeStruct((E,), jnp.float32),
        jax.ShapeDtypeStruct((E,), jnp.float32),
        jax.ShapeDtypeStruct((E,), jnp.float32),
    ),
    compiler_params=pltpu.CompilerParams(
        needs_layout_passes=False, use_tc_tiling_on_sc=False),
    scratch_types=[
        pltpu.VMEM_SHARED((N,), jnp.float32),
        pltpu.VMEM_SHARED((N,), jnp.float32),
        pltpu.VMEM_SHARED((N,), jnp.float32),
        pltpu.VMEM((B,), jnp.int32),
        pltpu.VMEM((B,), jnp.int32),
        pltpu.VMEM((B,), jnp.int32),
        pltpu.VMEM((B,), jnp.int32),
        pltpu.VMEM((B,), jnp.float32),
        pltpu.VMEM((B,), jnp.float32),
        pltpu.VMEM((B,), jnp.float32),
        pltpu.VMEM((B,), jnp.float32),
        pltpu.VMEM((B,), jnp.float32),
        pltpu.VMEM((B,), jnp.float32),
        pltpu.VMEM((B,), jnp.float32),
        pltpu.VMEM((B,), jnp.float32),
        pltpu.VMEM((B,), jnp.float32),
        pltpu.VMEM((B,), jnp.float32),
        pltpu.VMEM((B,), jnp.float32),
        pltpu.VMEM((B,), jnp.float32),
        pltpu.SemaphoreType.DMA,
        pltpu.SemaphoreType.DMA,
        pltpu.SemaphoreType.DMA,
        pltpu.SemaphoreType.DMA,
        pltpu.SemaphoreType.DMA,
        pltpu.SemaphoreType.DMA,
    ],
)
def _pairwise_sc(*refs):
    _body(*refs)


@jax.jit
def kernel(R, offsets, idx_i, idx_j):
    rx, ry, rz = R[:, 0], R[:, 1], R[:, 2]
    ii = idx_i.astype(jnp.int32)
    ij = idx_j.astype(jnp.int32)
    dx, dy, dz = _pairwise_sc(rx, ry, rz, ii, ij)
    return jnp.stack([dx, dy, dz], axis=-1) + offsets


# speedup vs baseline: 4.7309x; 1.0039x over previous
import functools

import jax
import jax.numpy as jnp
from jax import lax
from jax.experimental import pallas as pl
from jax.experimental.pallas import tpu as pltpu
from jax.experimental.pallas import tpu_sc as plsc

N = 50000
E = 1_600_000
NC = 2
NS = 16
NW = NC * NS
EPW = E // NW
BMAX = 3200
CHUNKS = [(k * BMAX, BMAX) for k in range(15)] + [(15 * BMAX, EPW - 15 * BMAX)]
NCH = len(CHUNKS)


def _body(rx_hbm, ry_hbm, rz_hbm, ii_hbm, ij_hbm,
          outx_hbm, outy_hbm, outz_hbm,
          tx, ty, tz,
          idx0, idx1,
          gx0, gy0, gz0, gx1, gy1, gz1,
          sin0, sin1, sg0, sg1, sout0, sout1):
    c = lax.axis_index("c")
    s = lax.axis_index("s")
    wid = s * NC + c

    @pl.when(s == 0)
    def _():
        pltpu.sync_copy(rx_hbm, tx)

    @pl.when(s == 1)
    def _():
        pltpu.sync_copy(ry_hbm, ty)

    @pl.when(s == 2)
    def _():
        pltpu.sync_copy(rz_hbm, tz)

    plsc.subcore_barrier()

    ebase = wid * EPW
    idxs = [(idx0, sin0), (idx1, sin1)]
    gs = [(gx0, gy0, gz0, sg0), (gx1, gy1, gz1, sg1)]
    souts = [sout0, sout1]

    def in_descs(ch):
        off, sz = CHUNKS[ch]
        idx_v, sem = idxs[ch % 2]
        sl = pl.ds(ebase + off, sz)
        return (pltpu.make_async_copy(ii_hbm.at[sl], idx_v.at[pl.ds(0, sz)], sem),
                pltpu.make_async_copy(ij_hbm.at[sl], idx_v.at[pl.ds(BMAX, sz)], sem))

    def g_descs(ch):
        _, sz = CHUNKS[ch]
        idx_v, _ = idxs[ch % 2]
        gx, gy, gz, sem = gs[ch % 2]
        return (
            pltpu.make_async_copy(tx.at[idx_v.at[pl.ds(0, sz)]], gx.at[pl.ds(0, sz)], sem),
            pltpu.make_async_copy(ty.at[idx_v.at[pl.ds(0, sz)]], gy.at[pl.ds(0, sz)], sem),
            pltpu.make_async_copy(tz.at[idx_v.at[pl.ds(0, sz)]], gz.at[pl.ds(0, sz)], sem),
            pltpu.make_async_copy(tx.at[idx_v.at[pl.ds(BMAX, sz)]], gx.at[pl.ds(BMAX, sz)], sem),
            pltpu.make_async_copy(ty.at[idx_v.at[pl.ds(BMAX, sz)]], gy.at[pl.ds(BMAX, sz)], sem),
            pltpu.make_async_copy(tz.at[idx_v.at[pl.ds(BMAX, sz)]], gz.at[pl.ds(BMAX, sz)], sem),
        )

    def out_descs(ch):
        off, sz = CHUNKS[ch]
        gx, gy, gz, _ = gs[ch % 2]
        sem = souts[ch % 2]
        sl = pl.ds(ebase + off, sz)
        return (pltpu.make_async_copy(gx.at[pl.ds(0, sz)], outx_hbm.at[sl], sem),
                pltpu.make_async_copy(gy.at[pl.ds(0, sz)], outy_hbm.at[sl], sem),
                pltpu.make_async_copy(gz.at[pl.ds(0, sz)], outz_hbm.at[sl], sem))

    def compute(ch):
        _, sz = CHUNKS[ch]
        gx, gy, gz, _ = gs[ch % 2]

        def ew(v, carry):
            vi = pl.ds(v * 16, 16)
            vj = pl.ds(BMAX + v * 16, 16)
            gx[vi] = gx[vj] - gx[vi]
            gy[vi] = gy[vj] - gy[vi]
            gz[vi] = gz[vj] - gz[vi]
            return carry

        lax.fori_loop(0, sz // 16, ew, 0, unroll=5)

    for d in in_descs(0):
        d.start()
    for d in in_descs(0):
        d.wait()
    for d in g_descs(0):
        d.start()
    for d in in_descs(1):
        d.start()

    for ch in range(NCH):
        if ch + 1 < NCH:
            for d in in_descs(ch + 1):
                d.wait()
            if ch >= 1:
                for d in out_descs(ch - 1):
                    d.wait()
            for d in g_descs(ch + 1):
                d.start()
        for d in g_descs(ch):
            d.wait()
        if ch + 2 < NCH:
            for d in in_descs(ch + 2):
                d.start()
        compute(ch)
        for d in out_descs(ch):
            d.start()

    for d in out_descs(NCH - 2):
        d.wait()
    for d in out_descs(NCH - 1):
        d.wait()


@functools.partial(
    pl.kernel,
    mesh=plsc.VectorSubcoreMesh(core_axis_name="c", subcore_axis_name="s"),
    out_type=(
        jax.ShapeDtypeStruct((E,), jnp.float32),
        jax.ShapeDtypeStruct((E,), jnp.float32),
        jax.ShapeDtypeStruct((E,), jnp.float32),
    ),
    compiler_params=pltpu.CompilerParams(
        needs_layout_passes=False, use_tc_tiling_on_sc=False),
    scratch_types=[
        pltpu.VMEM_SHARED((N,), jnp.float32),
        pltpu.VMEM_SHARED((N,), jnp.float32),
        pltpu.VMEM_SHARED((N,), jnp.float32),
        pltpu.VMEM((2 * BMAX,), jnp.int32),
        pltpu.VMEM((2 * BMAX,), jnp.int32),
        pltpu.VMEM((2 * BMAX,), jnp.float32),
        pltpu.VMEM((2 * BMAX,), jnp.float32),
        pltpu.VMEM((2 * BMAX,), jnp.float32),
        pltpu.VMEM((2 * BMAX,), jnp.float32),
        pltpu.VMEM((2 * BMAX,), jnp.float32),
        pltpu.VMEM((2 * BMAX,), jnp.float32),
        pltpu.SemaphoreType.DMA,
        pltpu.SemaphoreType.DMA,
        pltpu.SemaphoreType.DMA,
        pltpu.SemaphoreType.DMA,
        pltpu.SemaphoreType.DMA,
        pltpu.SemaphoreType.DMA,
    ],
)
def _pairwise_sc(*refs):
    _body(*refs)


@jax.jit
def kernel(R, offsets, idx_i, idx_j):
    rx, ry, rz = R[:, 0], R[:, 1], R[:, 2]
    ii = idx_i.astype(jnp.int32)
    ij = idx_j.astype(jnp.int32)
    dx, dy, dz = _pairwise_sc(rx, ry, rz, ii, ij)
    return jnp.stack([dx, dy, dz], axis=-1) + offsets
